# Initial kernel scaffold; baseline (speedup 1.0000x reference)
import jax
import jax.numpy as jnp
from jax.experimental import pallas as pl


def kernel(weight_e, weight_i):
    raise NotImplementedError("write your pallas kernel here")


# speedup vs baseline: 1.0354x; 1.0354x over previous
import jax
import jax.numpy as jnp
from jax.experimental import pallas as pl
from jax.experimental.pallas import tpu as pltpu

_N = 4096
_BLK = 512
_GRID = _N // _BLK


def _body(we_ref, wi_ref, oe_ref, oi_ref):
    i = pl.program_id(0)
    se = jnp.sum(jnp.square(jnp.minimum(we_ref[...], 0.0)))
    si = jnp.sum(jnp.square(jnp.minimum(wi_ref[...], 0.0)))

    @pl.when(i == 0)
    def _init():
        oe_ref[0, 0] = se
        oi_ref[0, 0] = si

    @pl.when(i > 0)
    def _acc():
        oe_ref[0, 0] += se
        oi_ref[0, 0] += si


def kernel(weight_e, weight_i):
    se, si = pl.pallas_call(
        _body,
        grid=(_GRID,),
        in_specs=[
            pl.BlockSpec((_BLK, _N), lambda i: (i, 0)),
            pl.BlockSpec((_BLK, _N), lambda i: (i, 0)),
        ],
        out_specs=[
            pl.BlockSpec((1, 1), lambda i: (0, 0), memory_space=pltpu.SMEM),
            pl.BlockSpec((1, 1), lambda i: (0, 0), memory_space=pltpu.SMEM),
        ],
        out_shape=[
            jax.ShapeDtypeStruct((1, 1), jnp.float32),
            jax.ShapeDtypeStruct((1, 1), jnp.float32),
        ],
    )(weight_e, weight_i)
    return jnp.sqrt(se[0, 0]) + jnp.sqrt(si[0, 0])
